# baseline (speedup 1.0000x reference)
import functools

import jax
import jax.numpy as jnp
from jax import lax
from jax.experimental import pallas as pl
from jax.experimental.pallas import tpu as pltpu
from jax.experimental.pallas import tpu_sc as plsc

B = 16
S_LEN = 128
N_HASH = 64
M_BLOOM = 1024
W_WIN = 3
NBLK = 2 * W_WIN + 1

LANES = 16
NUM_CORES = 2
NUM_SUBCORES = 16
MH = M_BLOOM // 2
SBLKS = S_LEN // LANES


def _hist_body(mh_hbm, cnt_hbm, inp, cnt, sem):
    wid = lax.axis_index("s") * NUM_CORES + lax.axis_index("c")
    b = wid // 2
    m_base = (wid % 2) * MH

    in_copy = pltpu.make_async_copy(mh_hbm.at[b], inp, sem)
    in_copy.start()

    zeros = jnp.zeros((LANES,), jnp.float32)

    def zrow(r, _):
        for j in range(S_LEN // LANES):
            cnt[r, pl.ds(j * LANES, LANES)] = zeros
        return 0

    lax.fori_loop(0, MH, zrow, 0)
    in_copy.wait()

    iota = lax.iota(jnp.int32, LANES)
    ones = jnp.ones((LANES,), jnp.float32)

    def scat(i, _):
        n = i // SBLKS
        sb = i - n * SBLKS
        s_vec = sb * LANES + iota
        n_vec = jnp.full((LANES,), n, jnp.int32)
        h = plsc.load_gather(inp, [s_vec, n_vec])
        rel = (h & (M_BLOOM - 1)) - m_base
        mask = (rel >= 0) & (rel < MH)
        rel_safe = jnp.where(mask, rel, 0)
        plsc.addupdate_scatter(cnt, [rel_safe, s_vec], ones, mask=mask)
        return 0

    lax.fori_loop(0, N_HASH * SBLKS, scat, 0)

    pltpu.sync_copy(cnt, cnt_hbm.at[b, pl.ds(m_base, MH), :])


def _sc_histogram(minhashes):
    mesh = plsc.VectorSubcoreMesh(
        core_axis_name="c", subcore_axis_name="s",
        num_cores=NUM_CORES, num_subcores=NUM_SUBCORES,
    )
    run = pl.kernel(
        _hist_body,
        out_type=jax.ShapeDtypeStruct((B, M_BLOOM, S_LEN), jnp.float32),
        mesh=mesh,
        scratch_types=[
            pltpu.VMEM((S_LEN, N_HASH), jnp.int32),
            pltpu.VMEM((MH, S_LEN), jnp.float32),
            pltpu.SemaphoreType.DMA,
        ],
        compiler_params=pltpu.CompilerParams(
            use_tc_tiling_on_sc=False, needs_layout_passes=False
        ),
    )
    return run(minhashes)


def _emit_body(cin, cout):
    x = cin[0]
    k = pl.program_id(1)

    def mk(kk):
        d = W_WIN - kk

        def br():
            if d > 0:
                blk = jnp.concatenate(
                    [jnp.zeros((M_BLOOM, d), jnp.float32), x[:, : S_LEN - d]],
                    axis=1,
                )
            elif d == 0:
                blk = x
            else:
                e = -d
                blk = jnp.concatenate(
                    [x[:, e:], jnp.zeros((M_BLOOM, e), jnp.float32)], axis=1
                )
            cout[0] = blk

        return br

    lax.switch(k, [mk(kk) for kk in range(NBLK)])


def _tc_emit(counts):
    return pl.pallas_call(
        _emit_body,
        out_shape=jax.ShapeDtypeStruct((B, NBLK * M_BLOOM, S_LEN), jnp.float32),
        grid=(B, NBLK),
        in_specs=[pl.BlockSpec((1, M_BLOOM, S_LEN), lambda i, k: (i, 0, 0))],
        out_specs=pl.BlockSpec((1, M_BLOOM, S_LEN), lambda i, k: (i, k, 0)),
    )(counts)


@functools.partial(jax.jit, static_argnames=())
def kernel(minhashes):
    return _tc_emit(_sc_histogram(minhashes))


# speedup vs baseline: 1.1345x; 1.1345x over previous
import functools

import jax
import jax.numpy as jnp
from jax import lax
from jax.experimental import pallas as pl
from jax.experimental.pallas import tpu as pltpu
from jax.experimental.pallas import tpu_sc as plsc

B = 16
S_LEN = 128
N_HASH = 64
M_BLOOM = 1024
W_WIN = 3
NBLK = 2 * W_WIN + 1

LANES = 16
NUM_CORES = 2
NUM_SUBCORES = 16
MH = M_BLOOM // 2
SBLKS = S_LEN // LANES


def _hist_body(mh_hbm, cnt_hbm, inp, cnt, sem):
    wid = lax.axis_index("s") * NUM_CORES + lax.axis_index("c")
    b = wid // 2
    m_base = (wid % 2) * MH

    in_copy = pltpu.make_async_copy(mh_hbm.at[b], inp, sem)
    in_copy.start()

    zeros = jnp.zeros((LANES,), jnp.float32)

    def zrow(r, _):
        for j in range(S_LEN // LANES):
            cnt[r, pl.ds(j * LANES, LANES)] = zeros
        return 0

    lax.fori_loop(0, MH, zrow, 0)
    in_copy.wait()

    iota = lax.iota(jnp.int32, LANES)
    ones = jnp.ones((LANES,), jnp.float32)

    def scat(i, _):
        n = i // SBLKS
        sb = i - n * SBLKS
        s_vec = sb * LANES + iota
        n_vec = jnp.full((LANES,), n, jnp.int32)
        h = plsc.load_gather(inp, [s_vec, n_vec])
        rel = (h & (M_BLOOM - 1)) - m_base
        mask = (rel >= 0) & (rel < MH)
        rel_safe = jnp.where(mask, rel, 0)
        plsc.addupdate_scatter(cnt, [rel_safe, s_vec], ones, mask=mask)
        return 0

    lax.fori_loop(0, N_HASH * SBLKS, scat, 0)

    pltpu.sync_copy(cnt, cnt_hbm.at[b, pl.ds(m_base, MH), :])


def _sc_histogram(minhashes):
    mesh = plsc.VectorSubcoreMesh(
        core_axis_name="c", subcore_axis_name="s",
        num_cores=NUM_CORES, num_subcores=NUM_SUBCORES,
    )
    run = pl.kernel(
        _hist_body,
        out_type=jax.ShapeDtypeStruct((B, M_BLOOM, S_LEN), jnp.float32),
        mesh=mesh,
        scratch_types=[
            pltpu.VMEM((S_LEN, N_HASH), jnp.int32),
            pltpu.VMEM((MH, S_LEN), jnp.float32),
            pltpu.SemaphoreType.DMA,
        ],
        compiler_params=pltpu.CompilerParams(
            use_tc_tiling_on_sc=False, needs_layout_passes=False
        ),
    )
    return run(minhashes)


def _shift_block(x, k):
    d = W_WIN - k
    if d > 0:
        return jnp.concatenate(
            [jnp.zeros((M_BLOOM, d), jnp.float32), x[:, : S_LEN - d]], axis=1
        )
    if d == 0:
        return x
    e = -d
    return jnp.concatenate(
        [x[:, e:], jnp.zeros((M_BLOOM, e), jnp.float32)], axis=1
    )


def _emit_body(cnt_hbm, out_hbm, xbuf, obuf, insems, outsems):
    def in_copy(b, slot):
        return pltpu.make_async_copy(cnt_hbm.at[b], xbuf.at[slot], insems.at[slot])

    def out_copy(b, k, slot):
        return pltpu.make_async_copy(
            obuf.at[slot],
            out_hbm.at[b, pl.ds(k * M_BLOOM, M_BLOOM), :],
            outsems.at[slot],
        )

    in_copy(0, 0).start()

    def do_pair(bb, _):
        for half in range(2):
            b = bb * 2 + half
            islot = half

            @pl.when(b + 1 < B)
            def _prefetch():
                in_copy(b + 1, 1 - islot).start()

            in_copy(b, islot).wait()
            x = xbuf[islot]
            for k in range(NBLK):
                oslot = (7 * half + k) % 2
                past = bb * 14 + 7 * half + k

                @pl.when(past >= 2)
                def _drain():
                    out_copy(0, 0, oslot).wait()

                obuf[oslot] = _shift_block(x, k)
                out_copy(b, k, oslot).start()
        return 0

    lax.fori_loop(0, B // 2, do_pair, 0)
    for oslot in range(2):
        out_copy(0, 0, oslot).wait()


def _tc_emit(counts):
    return pl.pallas_call(
        _emit_body,
        out_shape=jax.ShapeDtypeStruct((B, NBLK * M_BLOOM, S_LEN), jnp.float32),
        in_specs=[pl.BlockSpec(memory_space=pl.ANY)],
        out_specs=pl.BlockSpec(memory_space=pl.ANY),
        scratch_shapes=[
            pltpu.VMEM((2, M_BLOOM, S_LEN), jnp.float32),
            pltpu.VMEM((2, M_BLOOM, S_LEN), jnp.float32),
            pltpu.SemaphoreType.DMA((2,)),
            pltpu.SemaphoreType.DMA((2,)),
        ],
    )(counts)


@functools.partial(jax.jit, static_argnames=())
def kernel(minhashes):
    return _tc_emit(_sc_histogram(minhashes))


# speedup vs baseline: 1.7070x; 1.5046x over previous
import functools

import jax
import jax.numpy as jnp
from jax import lax
from jax.experimental import pallas as pl
from jax.experimental.pallas import tpu as pltpu
from jax.experimental.pallas import tpu_sc as plsc

B = 16
S_LEN = 128
N_HASH = 64
M_BLOOM = 1024
W_WIN = 3
NBLK = 2 * W_WIN + 1

LANES = 16
NUM_CORES = 2
NUM_SUBCORES = 16
MH = M_BLOOM // 2
SBLKS = S_LEN // LANES


def _hist_body(mh_hbm, cnt_hbm, inp, cnt, sem):
    wid = lax.axis_index("s") * NUM_CORES + lax.axis_index("c")
    b = wid // 2
    m_base = (wid % 2) * MH

    in_copy = pltpu.make_async_copy(mh_hbm.at[b], inp, sem)
    in_copy.start()

    zeros = jnp.zeros((LANES,), jnp.float32)

    def zrow(r, _):
        for j in range(S_LEN // LANES):
            cnt[r, pl.ds(j * LANES, LANES)] = zeros
        return 0

    lax.fori_loop(0, MH, zrow, 0)
    in_copy.wait()

    iota = lax.iota(jnp.int32, LANES)
    ones = jnp.ones((LANES,), jnp.float32)

    def scat(i, _):
        n = i // SBLKS
        sb = i - n * SBLKS
        s_vec = sb * LANES + iota
        n_vec = jnp.full((LANES,), n, jnp.int32)
        h = plsc.load_gather(inp, [s_vec, n_vec])
        rel = (h & (M_BLOOM - 1)) - m_base
        mask = (rel >= 0) & (rel < MH)
        rel_safe = jnp.where(mask, rel, 0)
        plsc.addupdate_scatter(cnt, [rel_safe, s_vec], ones, mask=mask)
        return 0

    lax.fori_loop(0, N_HASH * SBLKS, scat, 0)

    pltpu.sync_copy(cnt, cnt_hbm.at[b, pl.ds(m_base, MH), :])


def _sc_histogram(minhashes):
    mesh = plsc.VectorSubcoreMesh(
        core_axis_name="c", subcore_axis_name="s",
        num_cores=NUM_CORES, num_subcores=NUM_SUBCORES,
    )
    run = pl.kernel(
        _hist_body,
        out_type=jax.ShapeDtypeStruct((B, M_BLOOM, S_LEN), jnp.float32),
        mesh=mesh,
        scratch_types=[
            pltpu.VMEM((S_LEN, N_HASH), jnp.int32),
            pltpu.VMEM((MH, S_LEN), jnp.float32),
            pltpu.SemaphoreType.DMA,
        ],
        compiler_params=pltpu.CompilerParams(
            use_tc_tiling_on_sc=False, needs_layout_passes=False
        ),
    )
    return run(minhashes)


def _emit_body(cin, cout):
    x = cin[0]
    shifts = [W_WIN - k for k in range(NBLK) if k != 3]
    pm = jnp.concatenate(
        [jnp.eye(S_LEN, S_LEN, k=d, dtype=jnp.float32) for d in shifts], axis=1
    )
    y = jax.lax.dot_general(
        x, pm, (((1,), (0,)), ((), ())), preferred_element_type=jnp.float32
    )
    col = 0
    for k in range(NBLK):
        if k == 3:
            cout[0, 3 * M_BLOOM : 4 * M_BLOOM, :] = x
        else:
            cout[0, k * M_BLOOM : (k + 1) * M_BLOOM, :] = y[
                :, col * S_LEN : (col + 1) * S_LEN
            ]
            col += 1


def _tc_emit(counts):
    return pl.pallas_call(
        _emit_body,
        out_shape=jax.ShapeDtypeStruct((B, NBLK * M_BLOOM, S_LEN), jnp.float32),
        grid=(B,),
        in_specs=[pl.BlockSpec((1, M_BLOOM, S_LEN), lambda i: (i, 0, 0))],
        out_specs=pl.BlockSpec((1, NBLK * M_BLOOM, S_LEN), lambda i: (i, 0, 0)),
    )(counts)


@functools.partial(jax.jit, static_argnames=())
def kernel(minhashes):
    return _tc_emit(_sc_histogram(minhashes))


# speedup vs baseline: 1.8167x; 1.0643x over previous
import functools

import jax
import jax.numpy as jnp
from jax import lax
from jax.experimental import pallas as pl
from jax.experimental.pallas import tpu as pltpu
from jax.experimental.pallas import tpu_sc as plsc

B = 16
S_LEN = 128
N_HASH = 64
M_BLOOM = 1024
W_WIN = 3
NBLK = 2 * W_WIN + 1

LANES = 16
NUM_CORES = 2
NUM_SUBCORES = 16
MH = M_BLOOM // 2
SBLKS = S_LEN // LANES


SH = S_LEN // 2
SBLKS = SH // LANES


def _hist_body(mh_hbm, cnt_hbm, inp, cnt, sem):
    wid = lax.axis_index("s") * NUM_CORES + lax.axis_index("c")
    b = wid // 2
    s_base = (wid % 2) * SH

    in_copy = pltpu.make_async_copy(mh_hbm.at[b, pl.ds(s_base, SH), :], inp, sem)
    in_copy.start()

    zeros = jnp.zeros((LANES,), jnp.float32)

    def zrow(r, _):
        for j in range(SH // LANES):
            cnt[r, pl.ds(j * LANES, LANES)] = zeros
        return 0

    lax.fori_loop(0, M_BLOOM, zrow, 0)
    in_copy.wait()

    iota = lax.iota(jnp.int32, LANES)
    ones = jnp.ones((LANES,), jnp.float32)

    def scat(i, _):
        n = i // SBLKS
        sb = i - n * SBLKS
        s_vec = sb * LANES + iota
        n_vec = jnp.full((LANES,), n, jnp.int32)
        h = plsc.load_gather(inp, [s_vec, n_vec])
        m = h & (M_BLOOM - 1)
        plsc.addupdate_scatter(cnt, [m, s_vec], ones)
        return 0

    lax.fori_loop(0, N_HASH * SBLKS, scat, 0)

    pltpu.sync_copy(cnt, cnt_hbm.at[b, :, pl.ds(s_base, SH)])


def _sc_histogram(minhashes):
    mesh = plsc.VectorSubcoreMesh(
        core_axis_name="c", subcore_axis_name="s",
        num_cores=NUM_CORES, num_subcores=NUM_SUBCORES,
    )
    run = pl.kernel(
        _hist_body,
        out_type=jax.ShapeDtypeStruct((B, M_BLOOM, S_LEN), jnp.float32),
        mesh=mesh,
        scratch_types=[
            pltpu.VMEM((SH, N_HASH), jnp.int32),
            pltpu.VMEM((M_BLOOM, SH), jnp.float32),
            pltpu.SemaphoreType.DMA,
        ],
        compiler_params=pltpu.CompilerParams(
            use_tc_tiling_on_sc=False, needs_layout_passes=False
        ),
    )
    return run(minhashes)


def _emit_body(cin, cout):
    x = cin[0]
    shifts = [W_WIN - k for k in range(NBLK) if k != 3]
    pm = jnp.concatenate(
        [jnp.eye(S_LEN, S_LEN, k=d, dtype=jnp.float32) for d in shifts], axis=1
    )
    y = jax.lax.dot_general(
        x, pm, (((1,), (0,)), ((), ())), preferred_element_type=jnp.float32
    )
    col = 0
    for k in range(NBLK):
        if k == 3:
            cout[0, 3 * M_BLOOM : 4 * M_BLOOM, :] = x
        else:
            cout[0, k * M_BLOOM : (k + 1) * M_BLOOM, :] = y[
                :, col * S_LEN : (col + 1) * S_LEN
            ]
            col += 1


def _tc_emit(counts):
    return pl.pallas_call(
        _emit_body,
        out_shape=jax.ShapeDtypeStruct((B, NBLK * M_BLOOM, S_LEN), jnp.float32),
        grid=(B,),
        in_specs=[pl.BlockSpec((1, M_BLOOM, S_LEN), lambda i: (i, 0, 0))],
        out_specs=pl.BlockSpec((1, NBLK * M_BLOOM, S_LEN), lambda i: (i, 0, 0)),
    )(counts)


@functools.partial(jax.jit, static_argnames=())
def kernel(minhashes):
    return _tc_emit(_sc_histogram(minhashes))


# speedup vs baseline: 1.9286x; 1.0616x over previous
import functools

import jax
import jax.numpy as jnp
from jax import lax
from jax.experimental import pallas as pl
from jax.experimental.pallas import tpu as pltpu
from jax.experimental.pallas import tpu_sc as plsc

B = 16
S_LEN = 128
N_HASH = 64
M_BLOOM = 1024
W_WIN = 3
NBLK = 2 * W_WIN + 1

LANES = 16
NUM_CORES = 2
NUM_SUBCORES = 16
MH = M_BLOOM // 2
SBLKS = S_LEN // LANES


SH = S_LEN // 2
SBLKS = SH // LANES


def _hist_body(mh_hbm, cnt_hbm, inp, cnt, sem):
    wid = lax.axis_index("s") * NUM_CORES + lax.axis_index("c")
    b = wid // 2
    s_base = (wid % 2) * SH

    in_copy = pltpu.make_async_copy(mh_hbm.at[b, pl.ds(s_base, SH), :], inp, sem)
    in_copy.start()

    zeros = jnp.zeros((LANES,), jnp.float32)

    def zrow(r, _):
        for j in range(SH // LANES):
            cnt[r, pl.ds(j * LANES, LANES)] = zeros
        return 0

    lax.fori_loop(0, M_BLOOM, zrow, 0)
    in_copy.wait()

    iota = lax.iota(jnp.int32, LANES)
    ones = jnp.ones((LANES,), jnp.float32)

    def scat(i, _):
        n = i // SBLKS
        sb = i - n * SBLKS
        s_vec = sb * LANES + iota
        n_vec = jnp.full((LANES,), n, jnp.int32)
        h = plsc.load_gather(inp, [s_vec, n_vec])
        m = h & (M_BLOOM - 1)
        plsc.addupdate_scatter(cnt, [m, s_vec], ones)
        return 0

    lax.fori_loop(0, N_HASH * SBLKS, scat, 0)

    pltpu.sync_copy(cnt, cnt_hbm.at[b, :, pl.ds(s_base, SH)])


def _sc_histogram(minhashes):
    mesh = plsc.VectorSubcoreMesh(
        core_axis_name="c", subcore_axis_name="s",
        num_cores=NUM_CORES, num_subcores=NUM_SUBCORES,
    )
    run = pl.kernel(
        _hist_body,
        out_type=jax.ShapeDtypeStruct((B, M_BLOOM, S_LEN), jnp.float32),
        mesh=mesh,
        scratch_types=[
            pltpu.VMEM((SH, N_HASH), jnp.int32),
            pltpu.VMEM((M_BLOOM, SH), jnp.float32),
            pltpu.SemaphoreType.DMA,
        ],
        compiler_params=pltpu.CompilerParams(
            use_tc_tiling_on_sc=False, needs_layout_passes=False
        ),
    )
    return run(minhashes)


BPG = 4


def _emit_body(cin, cout):
    shifts = [W_WIN - k for k in range(NBLK) if k != 3]
    pm = jnp.concatenate(
        [jnp.eye(S_LEN, S_LEN, k=d, dtype=jnp.float32) for d in shifts], axis=1
    )
    for bb in range(BPG):
        x = cin[bb]
        y = jax.lax.dot_general(
            x, pm, (((1,), (0,)), ((), ())), preferred_element_type=jnp.float32
        )
        col = 0
        for k in range(NBLK):
            if k == 3:
                cout[bb, 3 * M_BLOOM : 4 * M_BLOOM, :] = x
            else:
                cout[bb, k * M_BLOOM : (k + 1) * M_BLOOM, :] = y[
                    :, col * S_LEN : (col + 1) * S_LEN
                ]
                col += 1


def _tc_emit(counts):
    return pl.pallas_call(
        _emit_body,
        out_shape=jax.ShapeDtypeStruct((B, NBLK * M_BLOOM, S_LEN), jnp.float32),
        grid=(B // BPG,),
        in_specs=[pl.BlockSpec((BPG, M_BLOOM, S_LEN), lambda i: (i, 0, 0))],
        out_specs=pl.BlockSpec((BPG, NBLK * M_BLOOM, S_LEN), lambda i: (i, 0, 0)),
        compiler_params=pltpu.CompilerParams(vmem_limit_bytes=120 * 1024 * 1024),
    )(counts)


@functools.partial(jax.jit, static_argnames=())
def kernel(minhashes):
    return _tc_emit(_sc_histogram(minhashes))
